# baseline (speedup 1.0000x reference)
import functools

import jax
import jax.numpy as jnp
from jax.experimental import pallas as pl
from jax.experimental.pallas import tpu as pltpu

_NSEM = 8


def _buffer_update_kernel(M, B, idx_smem, y_smem, by_smem, bt_smem, t_smem,
                          bx_any, x_any, out_bx, out_by, out_bt,
                          win_smem, sems):
    def _init(m, _):
        win_smem[m] = -1
        return 0
    jax.lax.fori_loop(0, M, _init, 0)

    def _scan(j, _):
        iv = idx_smem[j]

        @pl.when(iv < M)
        def _():
            win_smem[iv] = j
        return 0
    jax.lax.fori_loop(0, B, _scan, 0)

    t = t_smem[0]

    def _row(m, _):
        @pl.when(m >= _NSEM)
        def _():
            pltpu.make_async_copy(
                bx_any.at[m - _NSEM], out_bx.at[m - _NSEM],
                sems.at[(m - _NSEM) % _NSEM]).wait()

        j = win_smem[m]

        @pl.when(j >= 0)
        def _():
            pltpu.make_async_copy(
                x_any.at[j], out_bx.at[m], sems.at[m % _NSEM]).start()
            out_by[m] = y_smem[j]
            out_bt[m] = t

        @pl.when(j < 0)
        def _():
            pltpu.make_async_copy(
                bx_any.at[m], out_bx.at[m], sems.at[m % _NSEM]).start()
            out_by[m] = by_smem[m]
            out_bt[m] = bt_smem[m]
        return 0
    jax.lax.fori_loop(0, M, _row, 0)

    def _drain(i, _):
        m = M - _NSEM + i
        pltpu.make_async_copy(
            bx_any.at[m], out_bx.at[m], sems.at[m % _NSEM]).wait()
        return 0
    jax.lax.fori_loop(0, _NSEM, _drain, 0)


def kernel(bx, by, bt, x, y, indices, t):
    M = bx.shape[0]
    B = x.shape[0]
    t_arr = jnp.asarray(t, jnp.int32).reshape((1,))

    smem = functools.partial(pl.BlockSpec, memory_space=pltpu.SMEM)
    anys = functools.partial(pl.BlockSpec, memory_space=pltpu.MemorySpace.HBM)

    out_bx, out_by, out_bt = pl.pallas_call(
        functools.partial(_buffer_update_kernel, M, B),
        in_specs=[smem(), smem(), smem(), smem(), smem(), anys(), anys()],
        out_specs=[anys(), smem(), smem()],
        out_shape=[
            jax.ShapeDtypeStruct(bx.shape, bx.dtype),
            jax.ShapeDtypeStruct(by.shape, by.dtype),
            jax.ShapeDtypeStruct(bt.shape, bt.dtype),
        ],
        scratch_shapes=[
            pltpu.SMEM((M,), jnp.int32),
            pltpu.SemaphoreType.DMA((_NSEM,)),
        ],
    )(indices, y, by, bt, t_arr, bx, x)
    return (out_bx, out_by, out_bt)


# speedup vs baseline: 8.6709x; 8.6709x over previous
import functools

import jax
import jax.numpy as jnp
from jax.experimental import pallas as pl
from jax.experimental.pallas import tpu as pltpu


def _meta_kernel(M, B, idx_smem, y_smem, t_smem, by_ref, bt_ref,
                 win_out, by_out, bt_out):
    rows = by_ref.shape[0]
    m_mat = (jax.lax.broadcasted_iota(jnp.int32, (rows, 128), 0) * 128
             + jax.lax.broadcasted_iota(jnp.int32, (rows, 128), 1))

    def _scan(j, carry):
        win, byv = carry
        iv = idx_smem[j]
        hit = jnp.where(iv < M, iv, -1)
        eq = m_mat == hit
        win = jnp.where(eq, j, win)
        byv = jnp.where(eq, y_smem[j], byv)
        return (win, byv)

    win0 = jnp.full((rows, 128), -1, jnp.int32)
    win, byv = jax.lax.fori_loop(0, B, _scan, (win0, by_ref[...]))
    written = win >= 0
    win_out[...] = win
    by_out[...] = byv
    bt_out[...] = jnp.where(written, t_smem[0], bt_ref[...])


def _copy_kernel(win_ref, bx_ref, x_ref, out_ref):
    w = win_ref[pl.program_id(0)]

    @pl.when(w >= 0)
    def _():
        out_ref[...] = x_ref[...]

    @pl.when(w < 0)
    def _():
        out_ref[...] = bx_ref[...]


def kernel(bx, by, bt, x, y, indices, t):
    M = bx.shape[0]
    B = x.shape[0]
    row_shape = bx.shape[1:]
    t_arr = jnp.asarray(t, jnp.int32).reshape((1,))

    smem = functools.partial(pl.BlockSpec, memory_space=pltpu.SMEM)

    win2d, by2d, bt2d = pl.pallas_call(
        functools.partial(_meta_kernel, M, B),
        in_specs=[smem(), smem(), smem(),
                  pl.BlockSpec(memory_space=pltpu.VMEM),
                  pl.BlockSpec(memory_space=pltpu.VMEM)],
        out_specs=[pl.BlockSpec(memory_space=pltpu.VMEM)] * 3,
        out_shape=[jax.ShapeDtypeStruct((M // 128, 128), jnp.int32)] * 3,
    )(indices, y, t_arr, by.reshape(M // 128, 128), bt.reshape(M // 128, 128))

    win = win2d.reshape(M)
    new_by = by2d.reshape(M)
    new_bt = bt2d.reshape(M)

    blk = (1,) + row_shape
    zeros = (0,) * len(row_shape)
    grid_spec = pltpu.PrefetchScalarGridSpec(
        num_scalar_prefetch=1,
        grid=(M,),
        in_specs=[
            pl.BlockSpec(blk, lambda i, win_ref: (i,) + zeros),
            pl.BlockSpec(
                blk,
                lambda i, win_ref: (jnp.maximum(win_ref[i], 0),) + zeros),
        ],
        out_specs=pl.BlockSpec(blk, lambda i, win_ref: (i,) + zeros),
    )
    new_bx = pl.pallas_call(
        _copy_kernel,
        grid_spec=grid_spec,
        out_shape=jax.ShapeDtypeStruct(bx.shape, bx.dtype),
    )(win, bx, x)

    return (new_bx, new_by, new_bt)


# speedup vs baseline: 33.8298x; 3.9015x over previous
import functools

import jax
import jax.numpy as jnp
from jax.experimental import pallas as pl
from jax.experimental.pallas import tpu as pltpu

_CHUNK = 16


def _copy_body(bx_ref, out_ref):
    out_ref[...] = bx_ref[...]


def _scatter_body(M, B, cpy_any, x_any, idx_smem, y_smem, by_smem, bt_smem,
                  t_smem, out_bx, out_by, out_bt,
                  seen_smem, wj_smem, dst_smem, buf, sems):
    def _init(m, _):
        seen_smem[m] = 0
        out_by[m] = by_smem[m]
        out_bt[m] = bt_smem[m]
        return 0
    jax.lax.fori_loop(0, M, _init, 0)

    t = t_smem[0]

    def _scan(jr, cnt):
        j = B - 1 - jr
        iv = idx_smem[j]
        new = jnp.logical_and(iv < M, seen_smem[jnp.minimum(iv, M - 1)] == 0)

        @pl.when(new)
        def _():
            seen_smem[iv] = 1
            wj_smem[cnt] = j
            dst_smem[cnt] = iv
            out_by[iv] = y_smem[j]
            out_bt[iv] = t
        return cnt + jnp.where(new, 1, 0)
    cnt = jax.lax.fori_loop(0, B, _scan, 0)

    def _chunk(c, _):
        base = c * _CHUNK

        @pl.when(base < cnt)
        def _():
            for k in range(_CHUNK):
                e = base + k

                @pl.when(e < cnt)
                def _():
                    pltpu.make_async_copy(
                        x_any.at[wj_smem[e]], buf.at[k], sems.at[k]).start()
            for k in range(_CHUNK):
                e = base + k

                @pl.when(e < cnt)
                def _():
                    pltpu.make_async_copy(
                        x_any.at[wj_smem[e]], buf.at[k], sems.at[k]).wait()
            for k in range(_CHUNK):
                e = base + k

                @pl.when(e < cnt)
                def _():
                    pltpu.make_async_copy(
                        buf.at[k], out_bx.at[dst_smem[e]], sems.at[k]).start()
            for k in range(_CHUNK):
                e = base + k

                @pl.when(e < cnt)
                def _():
                    pltpu.make_async_copy(
                        buf.at[k], out_bx.at[dst_smem[e]], sems.at[k]).wait()
        return 0
    jax.lax.fori_loop(0, pl.cdiv(B, _CHUNK), _chunk, 0)


def kernel(bx, by, bt, x, y, indices, t):
    M = bx.shape[0]
    B = x.shape[0]
    row_shape = bx.shape[1:]
    t_arr = jnp.asarray(t, jnp.int32).reshape((1,))

    R = 16
    blk = (R,) + row_shape
    zeros = (0,) * len(row_shape)
    cpy = pl.pallas_call(
        _copy_body,
        grid=(M // R,),
        in_specs=[pl.BlockSpec(blk, lambda i: (i,) + zeros)],
        out_specs=pl.BlockSpec(blk, lambda i: (i,) + zeros),
        out_shape=jax.ShapeDtypeStruct(bx.shape, bx.dtype),
    )(bx)

    smem = functools.partial(pl.BlockSpec, memory_space=pltpu.SMEM)
    anys = functools.partial(pl.BlockSpec, memory_space=pltpu.MemorySpace.HBM)

    out_bx, out_by, out_bt = pl.pallas_call(
        functools.partial(_scatter_body, M, B),
        in_specs=[anys(), anys(), smem(), smem(), smem(), smem(), smem()],
        out_specs=[anys(), smem(), smem()],
        out_shape=[
            jax.ShapeDtypeStruct(bx.shape, bx.dtype),
            jax.ShapeDtypeStruct(by.shape, by.dtype),
            jax.ShapeDtypeStruct(bt.shape, bt.dtype),
        ],
        scratch_shapes=[
            pltpu.SMEM((M,), jnp.int32),
            pltpu.SMEM((B,), jnp.int32),
            pltpu.SMEM((B,), jnp.int32),
            pltpu.VMEM((_CHUNK,) + row_shape, bx.dtype),
            pltpu.SemaphoreType.DMA((_CHUNK,)),
        ],
        input_output_aliases={0: 0},
    )(cpy, x, indices, y, by, bt, t_arr)
    return (out_bx, out_by, out_bt)


# speedup vs baseline: 37.0624x; 1.0956x over previous
import functools

import jax
import jax.numpy as jnp
from jax.experimental import pallas as pl
from jax.experimental.pallas import tpu as pltpu

_R = 16
_NSLOT = 32


def _fused_body(M, B, idx_smem, y_smem, by_smem, bt_smem, t_smem, x_any,
                bx_ref, out_ref, out_by, out_bt,
                wj_smem, dst_smem, tmpj_smem, tmpd_smem,
                blkmask_smem, off_smem, curs_smem, buf, sems):
    i = pl.program_id(0)
    nblk = M // _R
    t = t_smem[0]

    @pl.when(i == 0)
    def _meta():
        def _z1(b, _):
            blkmask_smem[b] = 0
            return 0
        jax.lax.fori_loop(0, nblk, _z1, 0)

        def _z2(b, _):
            off_smem[b] = 0
            return 0
        jax.lax.fori_loop(0, nblk + 2, _z2, 0)

        def _scan(jr, cnt):
            j = B - 1 - jr
            iv = idx_smem[j]
            ivc = jnp.minimum(iv, M - 1)
            b = ivc // _R
            bit = jnp.int32(1) << (ivc % _R)
            new = jnp.logical_and(iv < M, (blkmask_smem[b] & bit) == 0)

            @pl.when(new)
            def _():
                blkmask_smem[b] = blkmask_smem[b] | bit
                tmpj_smem[cnt] = j
                tmpd_smem[cnt] = iv
                off_smem[b + 1] = off_smem[b + 1] + 1
            return cnt + jnp.where(new, 1, 0)
        cnt = jax.lax.fori_loop(0, B, _scan, 0)

        def _prefix(b, _):
            off_smem[b] = off_smem[b] + off_smem[b - 1]
            curs_smem[b - 1] = off_smem[b - 1]
            return 0
        jax.lax.fori_loop(1, nblk + 2, _prefix, 0)

        def _place(e, _):
            @pl.when(e < cnt)
            def _():
                d = tmpd_smem[e]
                b = d // _R
                p = curs_smem[b]
                curs_smem[b] = p + 1
                wj_smem[p] = tmpj_smem[e]
                dst_smem[p] = d
            return 0
        jax.lax.fori_loop(0, B, _place, 0)

        s0 = off_smem[0]
        n0 = off_smem[1] - s0
        for k in range(_R):
            @pl.when(k < n0)
            def _():
                e = s0 + k
                pltpu.make_async_copy(
                    x_any.at[wj_smem[e]], buf.at[e % _NSLOT],
                    sems.at[e % _NSLOT]).start()

    @pl.when(i + 1 < nblk)
    def _issue_next():
        s = off_smem[i + 1]
        n = off_smem[i + 2] - s
        for k in range(_R):
            @pl.when(k < n)
            def _():
                e = s + k
                pltpu.make_async_copy(
                    x_any.at[wj_smem[e]], buf.at[e % _NSLOT],
                    sems.at[e % _NSLOT]).start()

    out_ref[...] = bx_ref[...]

    mask = blkmask_smem[i]
    for k in range(_R):
        m = i * _R + k

        @pl.when(((mask >> k) & 1) == 0)
        def _():
            out_by[m] = by_smem[m]
            out_bt[m] = bt_smem[m]

    s = off_smem[i]
    n = off_smem[i + 1] - s
    for k in range(_R):
        @pl.when(k < n)
        def _():
            e = s + k
            slot = e % _NSLOT
            pltpu.make_async_copy(
                x_any.at[wj_smem[e]], buf.at[slot], sems.at[slot]).wait()
            d = dst_smem[e]
            out_ref[pl.ds(d - i * _R, 1)] = buf[pl.ds(slot, 1)]
            out_by[d] = y_smem[wj_smem[e]]
            out_bt[d] = t


def kernel(bx, by, bt, x, y, indices, t):
    M = bx.shape[0]
    B = x.shape[0]
    row_shape = bx.shape[1:]
    nblk = M // _R
    t_arr = jnp.asarray(t, jnp.int32).reshape((1,))

    smem = functools.partial(pl.BlockSpec, memory_space=pltpu.SMEM)
    anys = functools.partial(pl.BlockSpec, memory_space=pltpu.MemorySpace.HBM)

    blk = (_R,) + row_shape
    zeros = (0,) * len(row_shape)

    out_bx, out_by, out_bt = pl.pallas_call(
        functools.partial(_fused_body, M, B),
        grid=(nblk,),
        in_specs=[smem(), smem(), smem(), smem(), smem(), anys(),
                  pl.BlockSpec(blk, lambda i: (i,) + zeros)],
        out_specs=[pl.BlockSpec(blk, lambda i: (i,) + zeros), smem(), smem()],
        out_shape=[
            jax.ShapeDtypeStruct(bx.shape, bx.dtype),
            jax.ShapeDtypeStruct(by.shape, by.dtype),
            jax.ShapeDtypeStruct(bt.shape, bt.dtype),
        ],
        scratch_shapes=[
            pltpu.SMEM((B,), jnp.int32),
            pltpu.SMEM((B,), jnp.int32),
            pltpu.SMEM((B,), jnp.int32),
            pltpu.SMEM((B,), jnp.int32),
            pltpu.SMEM((nblk,), jnp.int32),
            pltpu.SMEM((nblk + 2,), jnp.int32),
            pltpu.SMEM((nblk + 1,), jnp.int32),
            pltpu.VMEM((_NSLOT,) + row_shape, bx.dtype),
            pltpu.SemaphoreType.DMA((_NSLOT,)),
        ],
    )(indices, y, by, bt, t_arr, x, bx)
    return (out_bx, out_by, out_bt)


# speedup vs baseline: 38.2381x; 1.0317x over previous
import functools

import jax
import jax.numpy as jnp
from jax.experimental import pallas as pl
from jax.experimental.pallas import tpu as pltpu

_R = 32
_NSLOT = 64


def _fused_body(M, B, idx_smem, y_smem, by_smem, bt_smem, t_smem, x_any,
                bx_ref, out_ref, out_by, out_bt,
                wj_smem, dst_smem, tmpj_smem, tmpd_smem,
                blkmask_smem, off_smem, curs_smem, buf, sems):
    i = pl.program_id(0)
    nblk = M // _R
    t = t_smem[0]

    @pl.when(i == 0)
    def _meta():
        def _z1(b, _):
            blkmask_smem[b] = 0
            return 0
        jax.lax.fori_loop(0, nblk, _z1, 0)

        def _z2(b, _):
            off_smem[b] = 0
            return 0
        jax.lax.fori_loop(0, nblk + 2, _z2, 0)

        def _scan(jr, cnt):
            j = B - 1 - jr
            iv = idx_smem[j]
            ivc = jnp.minimum(iv, M - 1)
            b = ivc // _R
            bit = jnp.int32(1) << (ivc % _R)
            new = jnp.logical_and(iv < M, (blkmask_smem[b] & bit) == 0)

            @pl.when(new)
            def _():
                blkmask_smem[b] = blkmask_smem[b] | bit
                tmpj_smem[cnt] = j
                tmpd_smem[cnt] = iv
                off_smem[b + 1] = off_smem[b + 1] + 1
            return cnt + jnp.where(new, 1, 0)
        cnt = jax.lax.fori_loop(0, B, _scan, 0)

        def _prefix(b, _):
            off_smem[b] = off_smem[b] + off_smem[b - 1]
            curs_smem[b - 1] = off_smem[b - 1]
            return 0
        jax.lax.fori_loop(1, nblk + 2, _prefix, 0)

        def _place(e, _):
            @pl.when(e < cnt)
            def _():
                d = tmpd_smem[e]
                b = d // _R
                p = curs_smem[b]
                curs_smem[b] = p + 1
                wj_smem[p] = tmpj_smem[e]
                dst_smem[p] = d
            return 0
        jax.lax.fori_loop(0, B, _place, 0)

        s0 = off_smem[0]
        n0 = off_smem[1] - s0
        for k in range(_R):
            @pl.when(k < n0)
            def _():
                e = s0 + k
                pltpu.make_async_copy(
                    x_any.at[wj_smem[e]], buf.at[e % _NSLOT],
                    sems.at[e % _NSLOT]).start()

    @pl.when(i + 1 < nblk)
    def _issue_next():
        s = off_smem[i + 1]
        n = off_smem[i + 2] - s
        for k in range(_R):
            @pl.when(k < n)
            def _():
                e = s + k
                pltpu.make_async_copy(
                    x_any.at[wj_smem[e]], buf.at[e % _NSLOT],
                    sems.at[e % _NSLOT]).start()

    out_ref[...] = bx_ref[...]

    mask = blkmask_smem[i]
    for k in range(_R):
        m = i * _R + k

        @pl.when(((mask >> k) & 1) == 0)
        def _():
            out_by[m] = by_smem[m]
            out_bt[m] = bt_smem[m]

    s = off_smem[i]
    n = off_smem[i + 1] - s
    for k in range(_R):
        @pl.when(k < n)
        def _():
            e = s + k
            slot = e % _NSLOT
            pltpu.make_async_copy(
                x_any.at[wj_smem[e]], buf.at[slot], sems.at[slot]).wait()
            d = dst_smem[e]
            out_ref[pl.ds(d - i * _R, 1)] = buf[pl.ds(slot, 1)]
            out_by[d] = y_smem[wj_smem[e]]
            out_bt[d] = t


def kernel(bx, by, bt, x, y, indices, t):
    M = bx.shape[0]
    B = x.shape[0]
    row_shape = bx.shape[1:]
    nblk = M // _R
    t_arr = jnp.asarray(t, jnp.int32).reshape((1,))

    smem = functools.partial(pl.BlockSpec, memory_space=pltpu.SMEM)
    anys = functools.partial(pl.BlockSpec, memory_space=pltpu.MemorySpace.HBM)

    blk = (_R,) + row_shape
    zeros = (0,) * len(row_shape)

    out_bx, out_by, out_bt = pl.pallas_call(
        functools.partial(_fused_body, M, B),
        grid=(nblk,),
        in_specs=[smem(), smem(), smem(), smem(), smem(), anys(),
                  pl.BlockSpec(blk, lambda i: (i,) + zeros)],
        out_specs=[pl.BlockSpec(blk, lambda i: (i,) + zeros), smem(), smem()],
        out_shape=[
            jax.ShapeDtypeStruct(bx.shape, bx.dtype),
            jax.ShapeDtypeStruct(by.shape, by.dtype),
            jax.ShapeDtypeStruct(bt.shape, bt.dtype),
        ],
        scratch_shapes=[
            pltpu.SMEM((B,), jnp.int32),
            pltpu.SMEM((B,), jnp.int32),
            pltpu.SMEM((B,), jnp.int32),
            pltpu.SMEM((B,), jnp.int32),
            pltpu.SMEM((nblk,), jnp.int32),
            pltpu.SMEM((nblk + 2,), jnp.int32),
            pltpu.SMEM((nblk + 1,), jnp.int32),
            pltpu.VMEM((_NSLOT,) + row_shape, bx.dtype),
            pltpu.SemaphoreType.DMA((_NSLOT,)),
        ],
    )(indices, y, by, bt, t_arr, x, bx)
    return (out_bx, out_by, out_bt)


# speedup vs baseline: 41.4718x; 1.0846x over previous
import functools

import jax
import jax.numpy as jnp
from jax.experimental import pallas as pl
from jax.experimental.pallas import tpu as pltpu

_R = 32
_NSLOT = 64


def _fused_body(M, B, idx_smem, y_smem, by_smem, bt_smem, t_smem, x_any,
                bx_ref, out_ref, out_by, out_bt,
                wjrow_smem, blkmask_smem, ctr_smem, buf, sems):
    i = pl.program_id(0)
    nblk = M // _R
    t = t_smem[0]

    def _issue_block(blk_id):
        mask = blkmask_smem[blk_id]
        for k in range(_R):
            @pl.when(((mask >> k) & 1) != 0)
            def _():
                m = blk_id * _R + k
                q = ctr_smem[0]
                pltpu.make_async_copy(
                    x_any.at[wjrow_smem[m]], buf.at[q % _NSLOT],
                    sems.at[q % _NSLOT]).start()
                ctr_smem[0] = q + 1

    @pl.when(i == 0)
    def _meta():
        def _z(b, _):
            blkmask_smem[b] = 0
            return 0
        jax.lax.fori_loop(0, nblk, _z, 0)
        ctr_smem[0] = 0
        ctr_smem[1] = 0

        def _scan(jr, _):
            j = B - 1 - jr
            iv = idx_smem[j]
            ivc = jnp.minimum(iv, M - 1)
            b = ivc // _R
            bit = jnp.int32(1) << (ivc % _R)

            @pl.when(jnp.logical_and(iv < M, (blkmask_smem[b] & bit) == 0))
            def _():
                blkmask_smem[b] = blkmask_smem[b] | bit
                wjrow_smem[ivc] = j
            return 0
        jax.lax.fori_loop(0, B, _scan, 0)

        _issue_block(0)

    @pl.when(i + 1 < nblk)
    def _issue_next():
        _issue_block(i + 1)

    out_ref[...] = bx_ref[...]

    mask = blkmask_smem[i]
    for k in range(_R):
        m = i * _R + k
        hit = ((mask >> k) & 1) != 0

        @pl.when(jnp.logical_not(hit))
        def _():
            out_by[m] = by_smem[m]
            out_bt[m] = bt_smem[m]

        @pl.when(hit)
        def _():
            p = ctr_smem[1]
            slot = p % _NSLOT
            pltpu.make_async_copy(
                x_any.at[wjrow_smem[m]], buf.at[slot], sems.at[slot]).wait()
            out_ref[pl.ds(k, 1)] = buf[pl.ds(slot, 1)]
            out_by[m] = y_smem[wjrow_smem[m]]
            out_bt[m] = t
            ctr_smem[1] = p + 1


def kernel(bx, by, bt, x, y, indices, t):
    M = bx.shape[0]
    B = x.shape[0]
    row_shape = bx.shape[1:]
    nblk = M // _R
    t_arr = jnp.asarray(t, jnp.int32).reshape((1,))

    smem = functools.partial(pl.BlockSpec, memory_space=pltpu.SMEM)
    anys = functools.partial(pl.BlockSpec, memory_space=pltpu.MemorySpace.HBM)

    blk = (_R,) + row_shape
    zeros = (0,) * len(row_shape)

    out_bx, out_by, out_bt = pl.pallas_call(
        functools.partial(_fused_body, M, B),
        grid=(nblk,),
        in_specs=[smem(), smem(), smem(), smem(), smem(), anys(),
                  pl.BlockSpec(blk, lambda i: (i,) + zeros)],
        out_specs=[pl.BlockSpec(blk, lambda i: (i,) + zeros), smem(), smem()],
        out_shape=[
            jax.ShapeDtypeStruct(bx.shape, bx.dtype),
            jax.ShapeDtypeStruct(by.shape, by.dtype),
            jax.ShapeDtypeStruct(bt.shape, bt.dtype),
        ],
        scratch_shapes=[
            pltpu.SMEM((M,), jnp.int32),
            pltpu.SMEM((nblk,), jnp.int32),
            pltpu.SMEM((2,), jnp.int32),
            pltpu.VMEM((_NSLOT,) + row_shape, bx.dtype),
            pltpu.SemaphoreType.DMA((_NSLOT,)),
        ],
    )(indices, y, by, bt, t_arr, x, bx)
    return (out_bx, out_by, out_bt)
